# baseline (speedup 1.0000x reference)
import functools

import jax
import jax.numpy as jnp
from jax import lax
from jax.experimental import pallas as pl
from jax.experimental.pallas import tpu as pltpu
from jax.experimental.pallas import tpu_sc as plsc

NUM_USER = 1000000
NUM_ITEM = 1000000
EMB_DIM = 32
BATCH = 16384

_NC, _NS, _L = 2, 16, 16
_NW = _NC * _NS
_CHUNK = BATCH // _NW
_GROUPS = _CHUNK // _L


def _rsqrt(x):
    i = plsc.bitcast(x, jnp.int32)
    i = 0x5F3759DF - (i >> 1)
    y = plsc.bitcast(i, jnp.float32)
    for _ in range(3):
        y = y * (1.5 - 0.5 * x * y * y)
    return y


def _scale(ns):
    norm = ns * _rsqrt(ns)
    return jnp.where(ns > 1.0, 1.0 / (norm + 1e-7), jnp.float32(1.0))


def _bpr_body(uidx_hbm, pidx_hbm, nidx_hbm, user_hbm, item_hbm, out_hbm,
              uidx_v, pidx_v, nidx_v, u_rows, p_rows, n_rows, out_v, sem):
    wid = lax.axis_index("s") * _NC + lax.axis_index("c")
    base = wid * _CHUNK

    pltpu.sync_copy(uidx_hbm.at[pl.ds(base, _CHUNK)], uidx_v)
    pltpu.sync_copy(pidx_hbm.at[pl.ds(base, _CHUNK)], pidx_v)
    pltpu.sync_copy(nidx_hbm.at[pl.ds(base, _CHUNK)], nidx_v)

    cu = pltpu.async_copy(user_hbm.at[uidx_v], u_rows, sem)
    cp = pltpu.async_copy(item_hbm.at[pidx_v], p_rows, sem)
    cn = pltpu.async_copy(item_hbm.at[nidx_v], n_rows, sem)
    cu.wait()
    cp.wait()
    cn.wait()

    lane = lax.iota(jnp.int32, _L)

    def group(g, _):
        rows = jnp.full((_L,), g * _L, jnp.int32) + lane
        zero = jnp.zeros((_L,), jnp.float32)
        ns_u = ns_p = ns_n = dp = dn = zero
        for d in range(EMB_DIM):
            col = jnp.full((_L,), d, jnp.int32)
            u = plsc.load_gather(u_rows, [rows, col])
            p = plsc.load_gather(p_rows, [rows, col])
            n = plsc.load_gather(n_rows, [rows, col])
            ns_u = ns_u + u * u
            ns_p = ns_p + p * p
            ns_n = ns_n + n * n
            dp = dp + u * p
            dn = dn + u * n
        su = _scale(ns_u)
        sp = _scale(ns_p)
        sn = _scale(ns_n)
        x = su * (sp * dp - sn * dn)
        out_v[pl.ds(g * _L, _L)] = 1.0 / (1.0 + jnp.exp(-x))
        return _

    lax.fori_loop(0, _GROUPS, group, 0, unroll=False)
    pltpu.sync_copy(out_v, out_hbm.at[pl.ds(base, _CHUNK)])


@functools.partial(jax.jit, static_argnames=())
def kernel(positive, negative, user_track_count, user_table, item_table):
    del user_track_count
    uidx = positive[:, 0].astype(jnp.int32)
    pidx = positive[:, 1].astype(jnp.int32)
    nidx = negative[:, 1].astype(jnp.int32)

    mesh = plsc.VectorSubcoreMesh(core_axis_name="c", subcore_axis_name="s")
    run = pl.kernel(
        _bpr_body,
        out_type=jax.ShapeDtypeStruct((BATCH,), jnp.float32),
        mesh=mesh,
        compiler_params=pltpu.CompilerParams(
            needs_layout_passes=False, use_tc_tiling_on_sc=False),
        scratch_types=[
            pltpu.VMEM((_CHUNK,), jnp.int32),
            pltpu.VMEM((_CHUNK,), jnp.int32),
            pltpu.VMEM((_CHUNK,), jnp.int32),
            pltpu.VMEM((_CHUNK, EMB_DIM), jnp.float32),
            pltpu.VMEM((_CHUNK, EMB_DIM), jnp.float32),
            pltpu.VMEM((_CHUNK, EMB_DIM), jnp.float32),
            pltpu.VMEM((_CHUNK,), jnp.float32),
            pltpu.SemaphoreType.DMA,
        ],
    )
    return run(uidx, pidx, nidx, user_table, item_table)
